# baseline (speedup 1.0000x reference)
import functools

import jax
import jax.numpy as jnp
from jax import lax
from jax.experimental import pallas as pl
from jax.experimental.pallas import tpu as pltpu
from jax.experimental.pallas import tpu_sc as plsc

TOKENS = 16384
HIDDEN = 4096
ROUTER = 64
EXPERTS = 64

BTT = 1024

NWORK = 32
LANES = 16
TPW = TOKENS // NWORK
NGROUP = TPW // LANES


def _tc_body(x_ref, w_in_ref, b_in_ref, bm_ref, w_gate_ref,
             b_gate_ref, lt_ref):
    w_in = w_in_ref[...]
    b_in = b_in_ref[...]
    bm = bm_ref[...]
    w_gate = w_gate_ref[...]
    b_gate = b_gate_ref[...]

    x_blk = x_ref[...]
    xp = jnp.dot(x_blk, w_in, preferred_element_type=jnp.float32) + b_in
    g = 0.1 * jnp.tanh(jnp.dot(xp, bm, preferred_element_type=jnp.float32))
    logits = jnp.dot(g, w_gate, preferred_element_type=jnp.float32) + b_gate
    lt_ref[0] = logits.T


def _tc_logits(x, W_in, b_in, Bm, W_gate, b_gate):
    b_in2 = b_in.reshape(1, ROUTER)
    b_gate2 = b_gate.reshape(1, EXPERTS)
    return pl.pallas_call(
        _tc_body,
        grid=(TOKENS // BTT,),
        in_specs=[
            pl.BlockSpec((BTT, HIDDEN), lambda i: (i, 0)),
            pl.BlockSpec((HIDDEN, ROUTER), lambda i: (0, 0)),
            pl.BlockSpec((1, ROUTER), lambda i: (0, 0)),
            pl.BlockSpec((ROUTER, ROUTER), lambda i: (0, 0)),
            pl.BlockSpec((ROUTER, EXPERTS), lambda i: (0, 0)),
            pl.BlockSpec((1, EXPERTS), lambda i: (0, 0)),
        ],
        out_specs=[
            pl.BlockSpec((1, EXPERTS, BTT), lambda i: (i, 0, 0)),
        ],
        out_shape=[
            jax.ShapeDtypeStruct((TOKENS // BTT, EXPERTS, BTT), jnp.float32),
        ],
    )(x, W_in, b_in2, Bm, W_gate, b_gate2)[0]


@functools.partial(
    pl.kernel,
    mesh=plsc.VectorSubcoreMesh(core_axis_name="c", subcore_axis_name="s"),
    out_type=[
        jax.ShapeDtypeStruct((NWORK, 2, TPW), jnp.int32),
        jax.ShapeDtypeStruct((NWORK, 2, TPW), jnp.float32),
    ],
    scratch_types=[
        pltpu.VMEM((EXPERTS, TPW), jnp.float32),
        pltpu.VMEM((2, TPW), jnp.int32),
        pltpu.VMEM((2, TPW), jnp.float32),
        pltpu.SemaphoreType.DMA,
    ],
)
def _sc_top2(lt_hbm, idx_hbm, w_hbm, tile_v, idx_v, w_v, sem):
    cid = lax.axis_index("c")
    sid = lax.axis_index("s")
    wid = sid * 2 + cid
    col0 = (wid % 2) * TPW
    half = TPW // 2
    cp2 = pltpu.make_async_copy(
        lt_hbm.at[wid // 2, :, pl.ds(col0 + half, half)],
        tile_v.at[:, pl.ds(half, half)], sem)
    cp2.start()
    pltpu.sync_copy(
        lt_hbm.at[wid // 2, :, pl.ds(col0, half)],
        tile_v.at[:, pl.ds(0, half)])

    def group(g, carry):
        bases = [g * (2 * LANES) + k * LANES for k in range(2)]
        m1 = [tile_v[0, pl.ds(b, LANES)] for b in bases]
        i1 = [jnp.zeros((LANES,), jnp.int32) for _ in bases]
        m2 = [jnp.full((LANES,), -jnp.inf, jnp.float32) for _ in bases]
        i2 = [jnp.zeros((LANES,), jnp.int32) for _ in bases]
        for e in range(1, EXPERTS):
            ev = jnp.full((LANES,), e, jnp.int32)
            for k, b in enumerate(bases):
                v = tile_v[e, pl.ds(b, LANES)]
                gt1 = v > m1[k]
                gt2 = v > m2[k]
                m2[k] = jnp.where(gt1, m1[k], jnp.where(gt2, v, m2[k]))
                i2[k] = jnp.where(gt1, i1[k], jnp.where(gt2, ev, i2[k]))
                m1[k] = jnp.where(gt1, v, m1[k])
                i1[k] = jnp.where(gt1, ev, i1[k])
        for k, b in enumerate(bases):
            ex = jnp.exp(m2[k] - m1[k])
            idx_v[0, pl.ds(b, LANES)] = i1[k]
            idx_v[1, pl.ds(b, LANES)] = i2[k]
            w_v[0, pl.ds(b, LANES)] = 1.0 / (1.0 + ex)
            w_v[1, pl.ds(b, LANES)] = ex / (1.0 + ex)
        return carry

    lax.fori_loop(0, NGROUP // 4, group, 0)
    cp2.wait()
    lax.fori_loop(NGROUP // 4, NGROUP // 2, group, 0)
    pltpu.sync_copy(idx_v, idx_hbm.at[wid])
    pltpu.sync_copy(w_v, w_hbm.at[wid])


def kernel(x, W_in, b_in, tau, A, Bm, W_gate, b_gate):
    del tau, A
    lt = _tc_logits(x, W_in, b_in, Bm, W_gate, b_gate)
    idx_t, w_t = _sc_top2(lt)
    idx = idx_t.transpose(0, 2, 1).reshape(TOKENS, 2)
    w = w_t.transpose(0, 2, 1).reshape(TOKENS, 2)
    return idx, w


# speedup vs baseline: 1.0042x; 1.0042x over previous
import functools

import jax
import jax.numpy as jnp
from jax import lax
from jax.experimental import pallas as pl
from jax.experimental.pallas import tpu as pltpu
from jax.experimental.pallas import tpu_sc as plsc

TOKENS = 16384
HIDDEN = 4096
ROUTER = 64
EXPERTS = 64

BTT = 1024

NWORK = 32
LANES = 16
TPW = TOKENS // NWORK
NGROUP = TPW // LANES


def _tc_body(x_ref, w_in_ref, b_in_ref, bm_ref, w_gate_ref,
             b_gate_ref, lt_ref):
    w_in = w_in_ref[...]
    b_in = b_in_ref[...]
    bm = bm_ref[...]
    w_gate = w_gate_ref[...]
    b_gate = b_gate_ref[...]

    x_blk = x_ref[...]
    xp = jnp.dot(x_blk, w_in, preferred_element_type=jnp.float32) + b_in
    g = 0.1 * jnp.tanh(jnp.dot(xp, bm, preferred_element_type=jnp.float32))
    logits = jnp.dot(g, w_gate, preferred_element_type=jnp.float32) + b_gate
    lt_ref[0] = logits.T


def _tc_logits(x, W_in, b_in, Bm, W_gate, b_gate):
    b_in2 = b_in.reshape(1, ROUTER)
    b_gate2 = b_gate.reshape(1, EXPERTS)
    return pl.pallas_call(
        _tc_body,
        grid=(TOKENS // BTT,),
        in_specs=[
            pl.BlockSpec((BTT, HIDDEN), lambda i: (i, 0)),
            pl.BlockSpec((HIDDEN, ROUTER), lambda i: (0, 0)),
            pl.BlockSpec((1, ROUTER), lambda i: (0, 0)),
            pl.BlockSpec((ROUTER, ROUTER), lambda i: (0, 0)),
            pl.BlockSpec((ROUTER, EXPERTS), lambda i: (0, 0)),
            pl.BlockSpec((1, EXPERTS), lambda i: (0, 0)),
        ],
        out_specs=[
            pl.BlockSpec((1, EXPERTS, BTT), lambda i: (i, 0, 0)),
        ],
        out_shape=[
            jax.ShapeDtypeStruct((TOKENS // BTT, EXPERTS, BTT), jnp.float32),
        ],
    )(x, W_in, b_in2, Bm, W_gate, b_gate2)[0]


@functools.partial(
    pl.kernel,
    mesh=plsc.VectorSubcoreMesh(core_axis_name="c", subcore_axis_name="s"),
    out_type=[
        jax.ShapeDtypeStruct((NWORK, 2, TPW), jnp.int32),
        jax.ShapeDtypeStruct((NWORK, 2, TPW), jnp.float32),
    ],
    scratch_types=[
        pltpu.VMEM((EXPERTS, TPW), jnp.float32),
        pltpu.VMEM((2, TPW), jnp.int32),
        pltpu.VMEM((2, TPW), jnp.float32),
    ],
)
def _sc_top2(lt_hbm, idx_hbm, w_hbm, tile_v, idx_v, w_v):
    cid = lax.axis_index("c")
    sid = lax.axis_index("s")
    wid = sid * 2 + cid
    pltpu.sync_copy(
        lt_hbm.at[wid // 2, :, pl.ds((wid % 2) * TPW, TPW)], tile_v)

    def group(g, carry):
        bases = [g * (2 * LANES) + k * LANES for k in range(2)]
        m1 = [tile_v[0, pl.ds(b, LANES)] for b in bases]
        i1 = [jnp.zeros((LANES,), jnp.int32) for _ in bases]
        m2 = [jnp.full((LANES,), -jnp.inf, jnp.float32) for _ in bases]
        i2 = [jnp.zeros((LANES,), jnp.int32) for _ in bases]
        for e in range(1, EXPERTS):
            ev = jnp.full((LANES,), e, jnp.int32)
            for k, b in enumerate(bases):
                v = tile_v[e, pl.ds(b, LANES)]
                gt1 = v > m1[k]
                gt2 = v > m2[k]
                m2[k] = jnp.where(gt1, m1[k], jnp.where(gt2, v, m2[k]))
                i2[k] = jnp.where(gt1, i1[k], jnp.where(gt2, ev, i2[k]))
                m1[k] = jnp.where(gt1, v, m1[k])
                i1[k] = jnp.where(gt1, ev, i1[k])
        for k, b in enumerate(bases):
            ex = jnp.exp(m2[k] - m1[k])
            idx_v[0, pl.ds(b, LANES)] = i1[k]
            idx_v[1, pl.ds(b, LANES)] = i2[k]
            w_v[0, pl.ds(b, LANES)] = 1.0 / (1.0 + ex)
            w_v[1, pl.ds(b, LANES)] = ex / (1.0 + ex)
        return carry

    lax.fori_loop(0, NGROUP // 2, group, 0)
    pltpu.sync_copy(idx_v, idx_hbm.at[wid])
    pltpu.sync_copy(w_v, w_hbm.at[wid])


def kernel(x, W_in, b_in, tau, A, Bm, W_gate, b_gate):
    del tau, A
    lt = _tc_logits(x, W_in, b_in, Bm, W_gate, b_gate)
    idx_t, w_t = _sc_top2(lt)
    idx = idx_t.transpose(0, 2, 1).reshape(TOKENS, 2)
    w = w_t.transpose(0, 2, 1).reshape(TOKENS, 2)
    return idx, w
